# Initial kernel scaffold; baseline (speedup 1.0000x reference)
import jax
import jax.numpy as jnp
from jax.experimental import pallas as pl


def kernel(inputs, pos_table):
    raise NotImplementedError("write your pallas kernel here")


# speedup vs baseline: 1.3737x; 1.3737x over previous
import jax
import jax.numpy as jnp
from jax.experimental import pallas as pl

SEQ_BLK = 1024


def _add_kernel(x_ref, t_ref, o_ref):
    o_ref[...] = x_ref[...] + t_ref[...]


def kernel(inputs, pos_table):
    batch, seq_len, dim = inputs.shape
    grid = (batch, seq_len // SEQ_BLK)
    return pl.pallas_call(
        _add_kernel,
        grid=grid,
        in_specs=[
            pl.BlockSpec((1, SEQ_BLK, dim), lambda b, s: (b, s, 0)),
            pl.BlockSpec((SEQ_BLK, dim), lambda b, s: (s, 0)),
        ],
        out_specs=pl.BlockSpec((1, SEQ_BLK, dim), lambda b, s: (b, s, 0)),
        out_shape=jax.ShapeDtypeStruct(inputs.shape, inputs.dtype),
    )(inputs, pos_table)


# speedup vs baseline: 1.8080x; 1.3161x over previous
import jax
import jax.numpy as jnp
from jax.experimental import pallas as pl

SEQ_BLK = 512


def _add_kernel(x_ref, t_ref, o_ref):
    o_ref[...] = x_ref[...] + t_ref[...][None]


def kernel(inputs, pos_table):
    batch, seq_len, dim = inputs.shape
    grid = (seq_len // SEQ_BLK,)
    return pl.pallas_call(
        _add_kernel,
        grid=grid,
        in_specs=[
            pl.BlockSpec((batch, SEQ_BLK, dim), lambda s: (0, s, 0)),
            pl.BlockSpec((SEQ_BLK, dim), lambda s: (s, 0)),
        ],
        out_specs=pl.BlockSpec((batch, SEQ_BLK, dim), lambda s: (0, s, 0)),
        out_shape=jax.ShapeDtypeStruct(inputs.shape, inputs.dtype),
    )(inputs, pos_table)
